# Initial kernel scaffold; baseline (speedup 1.0000x reference)
import jax
import jax.numpy as jnp
from jax.experimental import pallas as pl


def kernel(x, edge_index, W, b):
    raise NotImplementedError("write your pallas kernel here")


# speedup vs baseline: 6.0808x; 6.0808x over previous
import functools
import math

import jax
import jax.numpy as jnp
import numpy as np
from jax import lax
from jax.experimental import pallas as pl
from jax.experimental.pallas import tpu as pltpu
from jax.experimental.pallas import tpu_sc as plsc

N_NODES = 10000
D = 128
HALF = 64
D_POLY = 4

NC = 2
NS = 16
LANES = 16

N_PAD = 10240
ROWS_PER_TILE = N_PAD // NS
CHUNK = 128
CHUNKS_PER_TILE = 160
E_PAD = NS * CHUNKS_PER_TILE * CHUNK
ROW_CHUNKS = ROWS_PER_TILE // CHUNK
QW = HALF // LANES
K_STEPS = D_POLY + 2
IDX_HALVES = 2
CHUNKS_PER_HALF = CHUNKS_PER_TILE // IDX_HALVES


def _poly_coeffs(d):
  a = 1.4
  offset = 2
  thetas = []
  for i in range(offset, d + 1 + offset):
    m = d - i + offset
    beta = (math.gamma(i + 1) * math.gamma(d + 1 - i + offset)
            / math.gamma(d + 2 + offset))
    coeffs = np.zeros(d + offset + 1, dtype=np.float64)
    for j in range(m + 1):
      coeffs[i + j] += math.comb(m, j) * ((-1.0) ** j) / (a ** (i + j)) / (a * beta)
    thetas.append(coeffs)
  return np.stack(thetas)


_CHUNKS_PER_CORE = CHUNKS_PER_TILE // NC


def _deg_body(src_hbm, out_hbm, idx_v, ones_v, zero_v, tmp_v, deg_sp):
  c = lax.axis_index("c")
  s = lax.axis_index("s")
  base = s * ROWS_PER_TILE

  zero = jnp.zeros((LANES,), jnp.float32)
  one = jnp.ones((LANES,), jnp.float32)

  def fill(i, _):
    zero_v[pl.ds(i * LANES, LANES)] = zero
    return 0
  lax.fori_loop(0, ROWS_PER_TILE // LANES, fill, 0)
  for q in range(CHUNK // LANES):
    ones_v[pl.ds(q * LANES, LANES)] = one

  pltpu.sync_copy(zero_v, deg_sp.at[pl.ds(base, ROWS_PER_TILE)])
  plsc.subcore_barrier()

  pltpu.sync_copy(src_hbm.at[s, pl.ds(c * _CHUNKS_PER_CORE, _CHUNKS_PER_CORE)],
                  idx_v)

  def chunk(j, _):
    pltpu.sync_copy(ones_v, deg_sp.at[idx_v.at[j]], add=True)
    return 0
  lax.fori_loop(0, _CHUNKS_PER_CORE, chunk, 0)
  plsc.subcore_barrier()

  pltpu.sync_copy(deg_sp.at[pl.ds(base, ROWS_PER_TILE)], tmp_v)
  pltpu.sync_copy(tmp_v, out_hbm.at[c, pl.ds(base, ROWS_PER_TILE)])


_deg_kernel = pl.kernel(
    _deg_body,
    out_type=jax.ShapeDtypeStruct((NC, N_PAD), jnp.float32),
    mesh=plsc.VectorSubcoreMesh(core_axis_name="c", subcore_axis_name="s",
                                num_cores=NC, num_subcores=NS),
    compiler_params=pltpu.CompilerParams(use_tc_tiling_on_sc=False),
    scratch_types=[
        pltpu.VMEM((_CHUNKS_PER_CORE, CHUNK), jnp.int32),
        pltpu.VMEM((CHUNK,), jnp.float32),
        pltpu.VMEM((ROWS_PER_TILE,), jnp.float32),
        pltpu.VMEM((ROWS_PER_TILE,), jnp.float32),
        pltpu.VMEM_SHARED((N_PAD,), jnp.float32),
    ],
)


def _scalars_body(degp_ref, dinv_ref, dinv2_ref, drecip_ref):
  deg = jnp.sum(degp_ref[...], axis=0)
  r = lax.broadcasted_iota(jnp.int32, deg.shape, 0)
  l = lax.broadcasted_iota(jnp.int32, deg.shape, 1)
  valid = (r * 128 + l) < N_NODES
  degc = jnp.maximum(deg, 1.0)
  dinv = jnp.where(valid, lax.rsqrt(degc), 0.0)
  dinv_ref[...] = dinv
  dinv2_ref[...] = dinv * dinv
  drecip_ref[...] = jnp.sqrt(degc)


_scalars_call = pl.pallas_call(
    _scalars_body,
    out_shape=[jax.ShapeDtypeStruct((N_PAD // 128, 128), jnp.float32)] * 3,
)


_BLK = 2048


def _u0_body(x_ref, w_ref, b_ref, dinv_ref, u0_ref):
  h = jnp.dot(x_ref[...], w_ref[...], preferred_element_type=jnp.float32)
  h = h + b_ref[...]
  h = jnp.where(h >= 0, h, 0.01 * h)
  u = h * dinv_ref[...]
  u0_ref[0, :, :] = u[:, :HALF]
  u0_ref[1, :, :] = u[:, HALF:]


_u0_call = pl.pallas_call(
    _u0_body,
    grid=(N_PAD // _BLK,),
    in_specs=[
        pl.BlockSpec((_BLK, D), lambda i: (i, 0)),
        pl.BlockSpec((D, D), lambda i: (0, 0)),
        pl.BlockSpec((1, D), lambda i: (0, 0)),
        pl.BlockSpec((_BLK, 1), lambda i: (i, 0)),
    ],
    out_specs=pl.BlockSpec((NC, _BLK, HALF), lambda i: (0, i, 0)),
    out_shape=jax.ShapeDtypeStruct((NC, N_PAD, HALF), jnp.float32),
)


def _main_body(u0_hbm, dinv2_hbm, src_hbm, dst_hbm,
               o1, o2, o3, o4, o5, o6,
               idxs_v, idxd_v, dinv2_v, gbuf, abuf, zbuf,
               u_sp, agg):
  c = lax.axis_index("c")
  s = lax.axis_index("s")
  base = s * ROWS_PER_TILE

  pltpu.sync_copy(dinv2_hbm.at[pl.ds(base, ROWS_PER_TILE)], dinv2_v)

  zero = jnp.zeros((LANES,), jnp.float32)

  def zfill(i, _):
    for q in range(QW):
      zbuf[i, pl.ds(q * LANES, LANES)] = zero
    return 0
  lax.fori_loop(0, CHUNK, zfill, 0)

  def sinit(cc, _):
    r0 = base + cc * CHUNK
    pltpu.sync_copy(u0_hbm.at[c, pl.ds(r0, CHUNK)], gbuf)
    pltpu.sync_copy(gbuf, u_sp.at[pl.ds(r0, CHUNK)])
    pltpu.sync_copy(zbuf, agg.at[pl.ds(r0, CHUNK)])
    return 0
  lax.fori_loop(0, ROW_CHUNKS, sinit, 0)

  plsc.subcore_barrier()

  outs = [o1, o2, o3, o4, o5, o6]
  for k in range(K_STEPS):
    out_k = outs[k]

    for half in range(IDX_HALVES):
      pltpu.sync_copy(
          src_hbm.at[s, pl.ds(half * CHUNKS_PER_HALF, CHUNKS_PER_HALF)],
          idxs_v)
      pltpu.sync_copy(
          dst_hbm.at[s, pl.ds(half * CHUNKS_PER_HALF, CHUNKS_PER_HALF)],
          idxd_v)

      def phase_a(j, _):
        pltpu.sync_copy(u_sp.at[idxs_v.at[j]], gbuf)
        pltpu.sync_copy(gbuf, agg.at[idxd_v.at[j]], add=True)
        return 0
      lax.fori_loop(0, CHUNKS_PER_HALF, phase_a, 0)
    plsc.subcore_barrier()

    def phase_b(cc, _):
      r0 = base + cc * CHUNK
      pltpu.sync_copy(agg.at[pl.ds(r0, CHUNK)], abuf)
      pltpu.sync_copy(u_sp.at[pl.ds(r0, CHUNK)], gbuf)

      def rowu(r16, _):
        dvec = dinv2_v[pl.ds(cc * CHUNK + r16 * LANES, LANES)]
        for rr in range(LANES):
          d = dvec[rr]
          row = r16 * LANES + rr
          for q in range(QW):
            gbuf[row, pl.ds(q * LANES, LANES)] = (
                gbuf[row, pl.ds(q * LANES, LANES)]
                - d * abuf[row, pl.ds(q * LANES, LANES)])
        return 0
      lax.fori_loop(0, CHUNK // LANES, rowu, 0)

      pltpu.sync_copy(gbuf, u_sp.at[pl.ds(r0, CHUNK)])
      pltpu.sync_copy(zbuf, agg.at[pl.ds(r0, CHUNK)])
      pltpu.sync_copy(gbuf, out_k.at[c, pl.ds(r0, CHUNK)])
      return 0
    lax.fori_loop(0, ROW_CHUNKS, phase_b, 0)
    plsc.subcore_barrier()


_main_kernel = pl.kernel(
    _main_body,
    out_type=[jax.ShapeDtypeStruct((NC, N_PAD, HALF), jnp.float32)] * K_STEPS,
    mesh=plsc.VectorSubcoreMesh(core_axis_name="c", subcore_axis_name="s",
                                num_cores=NC, num_subcores=NS),
    compiler_params=pltpu.CompilerParams(use_tc_tiling_on_sc=False),
    scratch_types=[
        pltpu.VMEM((CHUNKS_PER_HALF, CHUNK), jnp.int32),
        pltpu.VMEM((CHUNKS_PER_HALF, CHUNK), jnp.int32),
        pltpu.VMEM((ROWS_PER_TILE,), jnp.float32),
        pltpu.VMEM((CHUNK, HALF), jnp.float32),
        pltpu.VMEM((CHUNK, HALF), jnp.float32),
        pltpu.VMEM((CHUNK, HALF), jnp.float32),
        pltpu.VMEM_SHARED((N_PAD, HALF), jnp.float32),
        pltpu.VMEM_SHARED((N_PAD, HALF), jnp.float32),
    ],
)


_CBLK = 1000
_THETA = _poly_coeffs(D_POLY)


def _combine_body(drecip_ref, *refs):
  u_halves = refs[:2 * (K_STEPS + 1)]
  out_ref = refs[2 * (K_STEPS + 1)]
  recip = drecip_ref[...]
  us = [r[...] for r in u_halves]
  for i in range(_THETA.shape[0]):
    for hf in range(2):
      acc = jnp.float32(_THETA[i, 0]) * us[hf]
      for k in range(1, K_STEPS + 1):
        acc = acc + jnp.float32(_THETA[i, k]) * us[2 * k + hf]
      out_ref[:, i * D + hf * HALF:i * D + (hf + 1) * HALF] = acc * recip


_combine_call = pl.pallas_call(
    _combine_body,
    grid=(N_NODES // _CBLK,),
    in_specs=[pl.BlockSpec((_CBLK, 1), lambda i: (i, 0))] + [
        pl.BlockSpec((_CBLK, HALF), lambda i: (i, 0))] * (2 * (K_STEPS + 1)),
    out_specs=pl.BlockSpec((_CBLK, 5 * D), lambda i: (i, 0)),
    out_shape=jax.ShapeDtypeStruct((N_NODES, 5 * D), jnp.float32),
)


@jax.jit
def kernel(x, edge_index, W, b):
  src = edge_index[0].astype(jnp.int32)
  dst = edge_index[1].astype(jnp.int32)

  n_fill = E_PAD - src.shape[0]
  pad_ids = N_NODES + (
      jax.lax.iota(jnp.int32, n_fill) % (N_PAD - N_NODES))
  src_p = jnp.concatenate([src, pad_ids]).reshape(NS, CHUNKS_PER_TILE, CHUNK)
  dst_p = jnp.concatenate([dst, pad_ids]).reshape(NS, CHUNKS_PER_TILE, CHUNK)

  x_p = jnp.pad(x, ((0, N_PAD - N_NODES), (0, 0)))

  degp = _deg_kernel(src_p)
  dinv, dinv2, drecip = _scalars_call(degp.reshape(NC, N_PAD // 128, 128))
  dinv_col = dinv.reshape(N_PAD, 1)
  u0 = _u0_call(x_p, W, b.reshape(1, D), dinv_col)

  u_steps = _main_kernel(u0, dinv2.reshape(N_PAD), src_p, dst_p)
  halves = []
  for u in [u0] + list(u_steps):
    halves.append(u[0])
    halves.append(u[1])
  return _combine_call(drecip.reshape(N_PAD, 1), *halves)


# speedup vs baseline: 8.0788x; 1.3286x over previous
import functools
import math

import jax
import jax.numpy as jnp
import numpy as np
from jax import lax
from jax.experimental import pallas as pl
from jax.experimental.pallas import tpu as pltpu
from jax.experimental.pallas import tpu_sc as plsc

N_NODES = 10000
D = 128
HALF = 64
D_POLY = 4

NC = 2
NS = 16
LANES = 16

N_PAD = 10240
ROWS_PER_TILE = N_PAD // NS
CHUNK = 128
CHUNKS_PER_TILE = 160
E_PAD = NS * CHUNKS_PER_TILE * CHUNK
ROW_CHUNKS = ROWS_PER_TILE // CHUNK
QW = HALF // LANES
K_STEPS = D_POLY + 2
IDX_HALVES = 4
CHUNKS_PER_HALF = CHUNKS_PER_TILE // IDX_HALVES


def _poly_coeffs(d):
  a = 1.4
  offset = 2
  thetas = []
  for i in range(offset, d + 1 + offset):
    m = d - i + offset
    beta = (math.gamma(i + 1) * math.gamma(d + 1 - i + offset)
            / math.gamma(d + 2 + offset))
    coeffs = np.zeros(d + offset + 1, dtype=np.float64)
    for j in range(m + 1):
      coeffs[i + j] += math.comb(m, j) * ((-1.0) ** j) / (a ** (i + j)) / (a * beta)
    thetas.append(coeffs)
  return np.stack(thetas)


_CHUNKS_PER_CORE = CHUNKS_PER_TILE // NC


def _deg_body(src_hbm, out_hbm, idx_v, ones_v, zero_v, tmp_v, deg_sp):
  c = lax.axis_index("c")
  s = lax.axis_index("s")
  base = s * ROWS_PER_TILE

  zero = jnp.zeros((LANES,), jnp.float32)
  one = jnp.ones((LANES,), jnp.float32)

  def fill(i, _):
    zero_v[pl.ds(i * LANES, LANES)] = zero
    return 0
  lax.fori_loop(0, ROWS_PER_TILE // LANES, fill, 0)
  for q in range(CHUNK // LANES):
    ones_v[pl.ds(q * LANES, LANES)] = one

  pltpu.sync_copy(zero_v, deg_sp.at[pl.ds(base, ROWS_PER_TILE)])
  plsc.subcore_barrier()

  pltpu.sync_copy(src_hbm.at[s, pl.ds(c * _CHUNKS_PER_CORE, _CHUNKS_PER_CORE)],
                  idx_v)

  def chunk(j, _):
    pltpu.sync_copy(ones_v, deg_sp.at[idx_v.at[j]], add=True)
    return 0
  lax.fori_loop(0, _CHUNKS_PER_CORE, chunk, 0)
  plsc.subcore_barrier()

  pltpu.sync_copy(deg_sp.at[pl.ds(base, ROWS_PER_TILE)], tmp_v)
  pltpu.sync_copy(tmp_v, out_hbm.at[c, pl.ds(base, ROWS_PER_TILE)])


_deg_kernel = pl.kernel(
    _deg_body,
    out_type=jax.ShapeDtypeStruct((NC, N_PAD), jnp.float32),
    mesh=plsc.VectorSubcoreMesh(core_axis_name="c", subcore_axis_name="s",
                                num_cores=NC, num_subcores=NS),
    compiler_params=pltpu.CompilerParams(use_tc_tiling_on_sc=False),
    scratch_types=[
        pltpu.VMEM((_CHUNKS_PER_CORE, CHUNK), jnp.int32),
        pltpu.VMEM((CHUNK,), jnp.float32),
        pltpu.VMEM((ROWS_PER_TILE,), jnp.float32),
        pltpu.VMEM((ROWS_PER_TILE,), jnp.float32),
        pltpu.VMEM_SHARED((N_PAD,), jnp.float32),
    ],
)


def _scalars_body(degp_ref, dinv_ref, dinv2_ref, drecip_ref):
  deg = jnp.sum(degp_ref[...], axis=0)
  r = lax.broadcasted_iota(jnp.int32, deg.shape, 0)
  l = lax.broadcasted_iota(jnp.int32, deg.shape, 1)
  valid = (r * 128 + l) < N_NODES
  degc = jnp.maximum(deg, 1.0)
  dinv = jnp.where(valid, lax.rsqrt(degc), 0.0)
  dinv_ref[...] = dinv
  dinv2_ref[...] = dinv * dinv
  drecip_ref[...] = jnp.sqrt(degc)


_scalars_call = pl.pallas_call(
    _scalars_body,
    out_shape=[jax.ShapeDtypeStruct((N_PAD // 128, 128), jnp.float32)] * 3,
)


_BLK = 2048


def _u0_body(x_ref, w_ref, b_ref, dinv_ref, u0_ref):
  h = jnp.dot(x_ref[...], w_ref[...], preferred_element_type=jnp.float32)
  h = h + b_ref[...]
  h = jnp.where(h >= 0, h, 0.01 * h)
  u = h * dinv_ref[...]
  u0_ref[0, :, :] = u[:, :HALF]
  u0_ref[1, :, :] = u[:, HALF:]


_u0_call = pl.pallas_call(
    _u0_body,
    grid=(N_PAD // _BLK,),
    in_specs=[
        pl.BlockSpec((_BLK, D), lambda i: (i, 0)),
        pl.BlockSpec((D, D), lambda i: (0, 0)),
        pl.BlockSpec((1, D), lambda i: (0, 0)),
        pl.BlockSpec((_BLK, 1), lambda i: (i, 0)),
    ],
    out_specs=pl.BlockSpec((NC, _BLK, HALF), lambda i: (0, i, 0)),
    out_shape=jax.ShapeDtypeStruct((NC, N_PAD, HALF), jnp.float32),
)


def _main_body(u0_hbm, dinv2_hbm, src_hbm, dst_hbm,
               o1, o2, o3, o4, o5, o6,
               idxs_v, idxd_v, dinv2_v, gbuf0, gbuf1, abuf, zbuf,
               sg0, sg1, ss0, ss1, sw,
               u_sp, agg):
  c = lax.axis_index("c")
  s = lax.axis_index("s")
  base = s * ROWS_PER_TILE

  pltpu.sync_copy(dinv2_hbm.at[pl.ds(base, ROWS_PER_TILE)], dinv2_v)

  zero = jnp.zeros((LANES,), jnp.float32)

  def zfill(i, _):
    for q in range(QW):
      zbuf[i, pl.ds(q * LANES, LANES)] = zero
    return 0
  lax.fori_loop(0, CHUNK, zfill, 0)

  def sinit(cc, _):
    r0 = base + cc * CHUNK
    pltpu.sync_copy(u0_hbm.at[c, pl.ds(r0, CHUNK)], gbuf0)
    pltpu.sync_copy(gbuf0, u_sp.at[pl.ds(r0, CHUNK)])
    pltpu.sync_copy(zbuf, agg.at[pl.ds(r0, CHUNK)])
    return 0
  lax.fori_loop(0, ROW_CHUNKS, sinit, 0)

  plsc.subcore_barrier()

  outs = [o1, o2, o3, o4, o5, o6]
  for k in range(K_STEPS):
    out_k = outs[k]

    for half in range(IDX_HALVES):
      pltpu.sync_copy(
          src_hbm.at[s, pl.ds(half * CHUNKS_PER_HALF, CHUNKS_PER_HALF)],
          idxs_v)
      pltpu.sync_copy(
          dst_hbm.at[s, pl.ds(half * CHUNKS_PER_HALF, CHUNKS_PER_HALF)],
          idxd_v)

      pltpu.sync_copy(u_sp.at[idxs_v.at[0]], gbuf0)

      def phase_a(it, _):
        j0 = 2 * it
        j1 = j0 + 1
        d_s0 = pltpu.async_copy(gbuf0, agg.at[idxd_v.at[j0]], ss0, add=True)
        d_g1 = pltpu.async_copy(u_sp.at[idxs_v.at[j1]], gbuf1, sg1)
        d_g1.wait()
        d_s0.wait()
        d_s1 = pltpu.async_copy(gbuf1, agg.at[idxd_v.at[j1]], ss1, add=True)

        @pl.when(it < CHUNKS_PER_HALF // 2 - 1)
        def _():
          pltpu.async_copy(u_sp.at[idxs_v.at[j0 + 2]], gbuf0, sg0).wait()
        d_s1.wait()
        return 0
      lax.fori_loop(0, CHUNKS_PER_HALF // 2, phase_a, 0)
    plsc.subcore_barrier()

    def phase_b(cc, _):
      r0 = base + cc * CHUNK
      d_a = pltpu.async_copy(agg.at[pl.ds(r0, CHUNK)], abuf, ss0)
      d_u = pltpu.async_copy(u_sp.at[pl.ds(r0, CHUNK)], gbuf0, sg0)
      d_a.wait()
      d_u.wait()

      def rowu(r16, _):
        dvec = dinv2_v[pl.ds(cc * CHUNK + r16 * LANES, LANES)]
        for rr in range(LANES):
          d = dvec[rr]
          row = r16 * LANES + rr
          for q in range(QW):
            gbuf0[row, pl.ds(q * LANES, LANES)] = (
                gbuf0[row, pl.ds(q * LANES, LANES)]
                - d * abuf[row, pl.ds(q * LANES, LANES)])
        return 0
      lax.fori_loop(0, CHUNK // LANES, rowu, 0)

      d1 = pltpu.async_copy(gbuf0, u_sp.at[pl.ds(r0, CHUNK)], sw)
      d2 = pltpu.async_copy(zbuf, agg.at[pl.ds(r0, CHUNK)], ss1)
      d3 = pltpu.async_copy(gbuf0, out_k.at[c, pl.ds(r0, CHUNK)], sg1)
      d1.wait()
      d2.wait()
      d3.wait()
      return 0
    lax.fori_loop(0, ROW_CHUNKS, phase_b, 0)
    plsc.subcore_barrier()


_main_kernel = pl.kernel(
    _main_body,
    out_type=[jax.ShapeDtypeStruct((NC, N_PAD, HALF), jnp.float32)] * K_STEPS,
    mesh=plsc.VectorSubcoreMesh(core_axis_name="c", subcore_axis_name="s",
                                num_cores=NC, num_subcores=NS),
    compiler_params=pltpu.CompilerParams(use_tc_tiling_on_sc=False),
    scratch_types=[
        pltpu.VMEM((CHUNKS_PER_HALF, CHUNK), jnp.int32),
        pltpu.VMEM((CHUNKS_PER_HALF, CHUNK), jnp.int32),
        pltpu.VMEM((ROWS_PER_TILE,), jnp.float32),
        pltpu.VMEM((CHUNK, HALF), jnp.float32),
        pltpu.VMEM((CHUNK, HALF), jnp.float32),
        pltpu.VMEM((CHUNK, HALF), jnp.float32),
        pltpu.VMEM((CHUNK, HALF), jnp.float32),
        pltpu.SemaphoreType.DMA,
        pltpu.SemaphoreType.DMA,
        pltpu.SemaphoreType.DMA,
        pltpu.SemaphoreType.DMA,
        pltpu.SemaphoreType.DMA,
        pltpu.VMEM_SHARED((N_PAD, HALF), jnp.float32),
        pltpu.VMEM_SHARED((N_PAD, HALF), jnp.float32),
    ],
)


_CBLK = 1000
_THETA = _poly_coeffs(D_POLY)


def _combine_body(drecip_ref, *refs):
  u_halves = refs[:2 * (K_STEPS + 1)]
  out_ref = refs[2 * (K_STEPS + 1)]
  recip = drecip_ref[...]
  us = [r[...] for r in u_halves]
  for i in range(_THETA.shape[0]):
    for hf in range(2):
      acc = jnp.float32(_THETA[i, 0]) * us[hf]
      for k in range(1, K_STEPS + 1):
        acc = acc + jnp.float32(_THETA[i, k]) * us[2 * k + hf]
      out_ref[:, i * D + hf * HALF:i * D + (hf + 1) * HALF] = acc * recip


_combine_call = pl.pallas_call(
    _combine_body,
    grid=(N_NODES // _CBLK,),
    in_specs=[pl.BlockSpec((_CBLK, 1), lambda i: (i, 0))] + [
        pl.BlockSpec((_CBLK, HALF), lambda i: (i, 0))] * (2 * (K_STEPS + 1)),
    out_specs=pl.BlockSpec((_CBLK, 5 * D), lambda i: (i, 0)),
    out_shape=jax.ShapeDtypeStruct((N_NODES, 5 * D), jnp.float32),
)


@jax.jit
def kernel(x, edge_index, W, b):
  src = edge_index[0].astype(jnp.int32)
  dst = edge_index[1].astype(jnp.int32)

  n_fill = E_PAD - src.shape[0]
  pad_ids = N_NODES + (
      jax.lax.iota(jnp.int32, n_fill) % (N_PAD - N_NODES))
  src_p = jnp.concatenate([src, pad_ids]).reshape(NS, CHUNKS_PER_TILE, CHUNK)
  dst_p = jnp.concatenate([dst, pad_ids]).reshape(NS, CHUNKS_PER_TILE, CHUNK)

  x_p = jnp.pad(x, ((0, N_PAD - N_NODES), (0, 0)))

  degp = _deg_kernel(src_p)
  dinv, dinv2, drecip = _scalars_call(degp.reshape(NC, N_PAD // 128, 128))
  dinv_col = dinv.reshape(N_PAD, 1)
  u0 = _u0_call(x_p, W, b.reshape(1, D), dinv_col)

  u_steps = _main_kernel(u0, dinv2.reshape(N_PAD), src_p, dst_p)
  halves = []
  for u in [u0] + list(u_steps):
    halves.append(u[0])
    halves.append(u[1])
  return _combine_call(drecip.reshape(N_PAD, 1), *halves)
